# Initial kernel scaffold; baseline (speedup 1.0000x reference)
import jax
import jax.numpy as jnp
from jax.experimental import pallas as pl


def kernel(x, edge_index, batch, W1, b1, W2, b2, W3, b3):
    raise NotImplementedError("write your pallas kernel here")


# speedup vs baseline: 29.5483x; 29.5483x over previous
import functools
import jax
import jax.numpy as jnp
from jax import lax
from jax.experimental import pallas as pl
from jax.experimental.pallas import tpu as pltpu
from jax.experimental.pallas import tpu_sc as plsc

_NC, _NS, _L = 2, 16, 16
_NW = _NC * _NS
_CH = 128
_BK = 16
_NUM_G = 128
_NEG = -3.0e38


def _cdiv(a, b):
    return (a + b - 1) // b


def _mesh():
    return plsc.VectorSubcoreMesh(core_axis_name="c", subcore_axis_name="s")


def _pad_edges(idx, ct, pad_val):
    e = idx.shape[0]
    return jnp.concatenate(
        [idx, jnp.full((ct * _CH - e,), pad_val, jnp.int32)]
    ).reshape(ct, _CH)


def _deg_pass(npad, kpw):
    nblk = kpw // _BK
    tsl = npad // _NS

    def body(dst_hbm, out_hbm, idxv, valv, zv, acc, sem):
        c = lax.axis_index("c")
        s = lax.axis_index("s")
        w = c * _NS + s

        def _z(i, carry):
            zv[pl.ds(i * _L, _L)] = jnp.zeros((_L,), jnp.float32)
            return carry

        lax.fori_loop(0, tsl // _L, _z, 0)
        for k in range(_CH // _L):
            valv[pl.ds(k * _L, _L)] = jnp.ones((_L,), jnp.float32)
        pltpu.sync_copy(zv, acc.at[pl.ds(s * tsl, tsl)])
        plsc.subcore_barrier()

        def _blk(b, carry):
            base = (w * kpw + b * _BK).astype(jnp.int32)
            pltpu.sync_copy(dst_hbm.at[pl.ds(base, _BK)], idxv)
            cps = [
                pltpu.async_copy(valv, acc.at[idxv.at[j]], sem, add=True)
                for j in range(_BK)
            ]
            for cp in cps:
                cp.wait()
            return carry

        lax.fori_loop(0, nblk, _blk, 0)
        plsc.subcore_barrier()
        pltpu.sync_copy(
            acc.at[pl.ds(s * tsl, tsl)], out_hbm.at[c, pl.ds(s * tsl, tsl)]
        )

    return pl.kernel(
        body,
        out_type=jax.ShapeDtypeStruct((_NC, npad), jnp.float32),
        mesh=_mesh(),
        scratch_types=[
            pltpu.VMEM((_BK, _CH), jnp.int32),
            pltpu.VMEM((_CH,), jnp.float32),
            pltpu.VMEM((tsl,), jnp.float32),
            pltpu.VMEM_SHARED((npad,), jnp.float32),
            pltpu.SemaphoreType.DMA,
        ],
    )


def _scalar_gs_pass(npad, kpw):
    nblk = kpw // _BK
    tsl = npad // _NS

    def body(tab_hbm, src_hbm, dst_hbm, out_hbm, srcv, dstv, gval, zv, acc,
             gsem, ssem):
        c = lax.axis_index("c")
        s = lax.axis_index("s")
        w = c * _NS + s

        def _z(i, carry):
            zv[pl.ds(i * _L, _L)] = jnp.zeros((_L,), jnp.float32)
            return carry

        lax.fori_loop(0, tsl // _L, _z, 0)
        pltpu.sync_copy(zv, acc.at[pl.ds(s * tsl, tsl)])
        plsc.subcore_barrier()

        def _blk(b, carry):
            base = (w * kpw + b * _BK).astype(jnp.int32)
            pltpu.sync_copy(src_hbm.at[pl.ds(base, _BK)], srcv)
            pltpu.sync_copy(dst_hbm.at[pl.ds(base, _BK)], dstv)
            gs = [
                pltpu.async_copy(tab_hbm.at[srcv.at[j]], gval.at[j], gsem)
                for j in range(_BK)
            ]
            for cp in gs:
                cp.wait()
            ss = [
                pltpu.async_copy(gval.at[j], acc.at[dstv.at[j]], ssem, add=True)
                for j in range(_BK)
            ]
            for cp in ss:
                cp.wait()
            return carry

        lax.fori_loop(0, nblk, _blk, 0)
        plsc.subcore_barrier()
        pltpu.sync_copy(
            acc.at[pl.ds(s * tsl, tsl)], out_hbm.at[c, pl.ds(s * tsl, tsl)]
        )

    return pl.kernel(
        body,
        out_type=jax.ShapeDtypeStruct((_NC, npad), jnp.float32),
        mesh=_mesh(),
        scratch_types=[
            pltpu.VMEM((_BK, _CH), jnp.int32),
            pltpu.VMEM((_BK, _CH), jnp.int32),
            pltpu.VMEM((_BK, _CH), jnp.float32),
            pltpu.VMEM((tsl,), jnp.float32),
            pltpu.VMEM_SHARED((npad,), jnp.float32),
            pltpu.SemaphoreType.DMA,
            pltpu.SemaphoreType.DMA,
        ],
    )


_BKC = 8


def _edge_pass(npad, kpw):
    nblk = kpw // _BKC
    tsl = npad // _NS
    zrows = tsl // 16

    def body(tab_hbm, src_hbm, dst_hbm, out_hbm, srcv, dstv, rows, zv, acc,
             gsem, ssem):
        c = lax.axis_index("c")
        s = lax.axis_index("s")
        off = (c * npad).astype(jnp.int32)

        def _z(i, carry):
            zv[i, :] = jnp.zeros((_L,), jnp.float32)
            return carry

        lax.fori_loop(0, zrows, _z, 0)
        for r in range(16):
            pltpu.sync_copy(zv, acc.at[pl.ds(s * tsl + r * zrows, zrows)])
        plsc.subcore_barrier()

        def _blk(b, carry):
            base = (s * kpw + b * _BKC).astype(jnp.int32)
            pltpu.sync_copy(src_hbm.at[pl.ds(base, _BKC)], srcv)
            pltpu.sync_copy(dst_hbm.at[pl.ds(base, _BKC)], dstv)
            for j in range(_BKC):
                for k in range(_CH // _L):
                    sl = pl.ds(k * _L, _L)
                    srcv[j, sl] = srcv[j, sl] + off
            gs = [
                pltpu.async_copy(tab_hbm.at[srcv.at[j]], rows.at[j], gsem)
                for j in range(_BKC)
            ]
            for cp in gs:
                cp.wait()
            ss = [
                pltpu.async_copy(rows.at[j], acc.at[dstv.at[j]], ssem, add=True)
                for j in range(_BKC)
            ]
            for cp in ss:
                cp.wait()
            return carry

        lax.fori_loop(0, nblk, _blk, 0)
        plsc.subcore_barrier()
        pltpu.sync_copy(
            acc.at[pl.ds(s * tsl, tsl)], out_hbm.at[c, pl.ds(s * tsl, tsl)]
        )

    return pl.kernel(
        body,
        out_type=jax.ShapeDtypeStruct((_NC, npad, _L), jnp.float32),
        mesh=_mesh(),
        compiler_params=pltpu.CompilerParams(use_tc_tiling_on_sc=False),
        scratch_types=[
            pltpu.VMEM((_BKC, _CH), jnp.int32),
            pltpu.VMEM((_BKC, _CH), jnp.int32),
            pltpu.VMEM((_BKC, _CH, _L), jnp.float32),
            pltpu.VMEM((tsl // 16, _L), jnp.float32),
            pltpu.VMEM_SHARED((npad, _L), jnp.float32),
            pltpu.SemaphoreType.DMA,
            pltpu.SemaphoreType.DMA,
        ],
    )


def _pool_pass(npad, gpw, cpr):
    def body(h2_hbm, starts_hbm, out_hbm, sv, hv, ob):
        c = lax.axis_index("c")
        s = lax.axis_index("s")
        w = c * _NS + s
        pltpu.sync_copy(starts_hbm, sv)
        zero = jnp.zeros((_L,), jnp.float32)
        bounds = sv[pl.ds(w * gpw, _L)]
        for gi in range(gpw):
            lo = bounds[gi]
            hi = bounds[gi + 1]
            lo8 = (lo // 8) * 8
            nch = lax.div(hi - lo8 + (cpr - 1), cpr)

            def _chunk(ci, carry):
                m0, m1 = carry
                base = lo8 + ci * cpr
                pltpu.sync_copy(h2_hbm.at[pl.ds(base, cpr)], hv)
                for r in range(cpr):
                    pos = base + r
                    ok = jnp.logical_and(pos >= lo, pos < hi)
                    okf = jnp.where(ok, 1.0, 0.0).astype(jnp.float32)
                    v0 = hv[r, pl.ds(0, _L)]
                    v1 = hv[r, pl.ds(_L, _L)]
                    m0 = jnp.maximum(m0, v0 * okf)
                    m1 = jnp.maximum(m1, v1 * okf)
                return m0, m1

            m0, m1 = lax.fori_loop(0, nch, _chunk, (zero, zero))
            ob[gi, pl.ds(0, _L)] = m0
            ob[gi, pl.ds(_L, _L)] = m1
        pltpu.sync_copy(ob, out_hbm.at[pl.ds(w * gpw, gpw)])

    return pl.kernel(
        body,
        out_type=jax.ShapeDtypeStruct((_NUM_G, 2 * _L), jnp.float32),
        mesh=_mesh(),
        scratch_types=[
            pltpu.VMEM((_NUM_G + 16,), jnp.int32),
            pltpu.VMEM((cpr, 2 * _L), jnp.float32),
            pltpu.VMEM((gpw, 2 * _L), jnp.float32),
        ],
    )


def _tc_stage1(npad, blk):
    grid = npad // blk

    def body(p0_ref, p1_ref, x_ref, dinv_ref, q1_ref):
        deg = 1.0 + p0_ref[...] + p1_ref[...]
        dinv = lax.rsqrt(deg)
        dinv_ref[...] = dinv
        q1_ref[...] = dinv * x_ref[...]

    col = lambda i: (i, 0)
    return pl.pallas_call(
        body,
        grid=(grid,),
        in_specs=[
            pl.BlockSpec((blk, 1), col),
            pl.BlockSpec((blk, 1), col),
            pl.BlockSpec((blk, 1), col),
        ],
        out_specs=[
            pl.BlockSpec((blk, 1), col),
            pl.BlockSpec((blk, 1), col),
        ],
        out_shape=[
            jax.ShapeDtypeStruct((npad, 1), jnp.float32),
            jax.ShapeDtypeStruct((npad, 1), jnp.float32),
        ],
    )


def _tc_stage2(npad, blk):
    grid = npad // blk

    def body(s0_ref, s1_ref, q1_ref, dinv_ref, w1_ref, b1_ref, w2_ref, b2_ref,
             out_ref):
        dinv = dinv_ref[...]
        s1 = dinv * (q1_ref[...] + s0_ref[...] + s1_ref[...])
        h1 = jnp.maximum(s1 * w1_ref[...] + b1_ref[...], 0.0)
        p2 = dinv * jnp.dot(h1, w2_ref[...],
                            preferred_element_type=jnp.float32)
        out_ref[0] = p2[:, : _L]
        out_ref[1] = p2[:, _L:]

    col = lambda i: (i, 0)
    return pl.pallas_call(
        body,
        grid=(grid,),
        in_specs=[
            pl.BlockSpec((blk, 1), col),
            pl.BlockSpec((blk, 1), col),
            pl.BlockSpec((blk, 1), col),
            pl.BlockSpec((blk, 1), col),
            pl.BlockSpec((1, 32), lambda i: (0, 0)),
            pl.BlockSpec((1, 32), lambda i: (0, 0)),
            pl.BlockSpec((32, 32), lambda i: (0, 0)),
            pl.BlockSpec((1, 32), lambda i: (0, 0)),
        ],
        out_specs=[pl.BlockSpec((2, blk, _L), lambda i: (0, i, 0))],
        out_shape=[jax.ShapeDtypeStruct((2, npad, _L), jnp.float32)],
    )


def _tc_stage3(npad, blk):
    grid = npad // blk

    def body(acc_ref, p2_ref, dinv_ref, b2_ref, h2_ref):
        t = acc_ref[...] + p2_ref[...]
        cat = jnp.concatenate([t[0], t[1]], axis=-1)
        h2_ref[...] = jnp.maximum(dinv_ref[...] * cat + b2_ref[...], 0.0)

    return pl.pallas_call(
        body,
        grid=(grid,),
        in_specs=[
            pl.BlockSpec((2, blk, _L), lambda i: (0, i, 0)),
            pl.BlockSpec((2, blk, _L), lambda i: (0, i, 0)),
            pl.BlockSpec((blk, 1), lambda i: (i, 0)),
            pl.BlockSpec((1, 32), lambda i: (0, 0)),
        ],
        out_specs=pl.BlockSpec((blk, 32), lambda i: (i, 0)),
        out_shape=jax.ShapeDtypeStruct((npad, 32), jnp.float32),
    )


def _tc_final():
    def body(g_ref, w3_ref, b3_ref, out_ref):
        z = jnp.dot(g_ref[...], w3_ref[...],
                    preferred_element_type=jnp.float32) + b3_ref[...]
        m = jnp.max(z, axis=1, keepdims=True)
        ez = jnp.exp(z - m)
        out_ref[...] = ez / jnp.sum(ez, axis=1, keepdims=True)

    return pl.pallas_call(
        body,
        out_shape=jax.ShapeDtypeStruct((_NUM_G, 2), jnp.float32),
    )


def kernel(x, edge_index, batch, W1, b1, W2, b2, W3, b3):
    n = x.shape[0]
    e = edge_index.shape[1]
    npad = _cdiv(n + 8, 2048) * 2048
    pad_idx = npad - 8
    blk = 2048

    src, dst = edge_index[0], edge_index[1]

    kpw_s = _cdiv(e, _CH * _NW * _BK) * _BK
    src_s = _pad_edges(src, kpw_s * _NW, pad_idx)
    dst_s = _pad_edges(dst, kpw_s * _NW, pad_idx)
    kpw_h = _cdiv(e, _CH * _NS * _BKC) * _BKC
    src_h = _pad_edges(src, kpw_h * _NS, pad_idx)
    dst_h = _pad_edges(dst, kpw_h * _NS, pad_idx)

    xp = jnp.zeros((npad, 1), jnp.float32).at[:n].set(x)

    part = _deg_pass(npad, kpw_s)(dst_s)
    dinv_p, q1_p = _tc_stage1(npad, blk)(
        part[0].reshape(npad, 1), part[1].reshape(npad, 1), xp
    )

    s1part = _scalar_gs_pass(npad, kpw_s)(q1_p.reshape(npad), src_s, dst_s)
    (p2cat,) = _tc_stage2(npad, blk)(
        s1part[0].reshape(npad, 1), s1part[1].reshape(npad, 1), q1_p, dinv_p,
        W1.reshape(1, 32), b1.reshape(1, 32), W2, b2.reshape(1, 32),
    )

    acc2 = _edge_pass(npad, kpw_h)(p2cat.reshape(2 * npad, _L), src_h, dst_h)
    h2 = _tc_stage3(npad, blk)(acc2, p2cat, dinv_p, b2.reshape(1, 32))

    starts = jnp.searchsorted(
        batch, jnp.arange(_NUM_G + 1, dtype=jnp.int32)
    ).astype(jnp.int32)
    startsp = jnp.concatenate(
        [starts, jnp.full((15,), n, jnp.int32)]
    )
    gmax = _pool_pass(npad, _NUM_G // _NW, 64)(h2, startsp)
    return _tc_final()(gmax, W3, b3.reshape(1, 2))
